# Initial kernel scaffold; baseline (speedup 1.0000x reference)
import jax
import jax.numpy as jnp
from jax.experimental import pallas as pl


def kernel(x, c1W, c1D, c2W, c2D, c3W, c3D, c4W, c4D, c5W, c5D, l1W, l1D, l2W, l2D, l3W):
    raise NotImplementedError("write your pallas kernel here")


# speedup vs baseline: 1.2941x; 1.2941x over previous
import functools

import jax
import jax.numpy as jnp
from jax.experimental import pallas as pl

_EPS = 1e-6
_BN_EPS = 1e-5
_K = 20
_KPAD = 32
_NEG = -3.0e38


def _knn_kernel(xt_ref, idx_ref, *, n, k):
    xt = xt_ref[...]
    inner = jax.lax.dot_general(
        xt, xt, (((1,), (1,)), ((), ())), preferred_element_type=jnp.float32
    )
    xx = jnp.sum(xt * xt, axis=1, keepdims=True)
    pd = 2.0 * inner - xx - jnp.transpose(xx)
    col = jax.lax.broadcasted_iota(jnp.int32, (n, n), 1)
    lane = jax.lax.broadcasted_iota(jnp.int32, (n, _KPAD), 1)
    acc = jnp.zeros((n, _KPAD), dtype=jnp.int32)
    for j in range(k):
        m = jnp.max(pd, axis=1, keepdims=True)
        idx = jnp.min(jnp.where(pd >= m, col, n), axis=1, keepdims=True)
        acc = jnp.where(lane == j, idx, acc)
        pd = jnp.where(col == idx, _NEG, pd)
    idx_ref[...] = acc


def _knn_idx(xf):
    b, c, n = xf.shape
    cpad = max(128, ((c + 127) // 128) * 128)
    xt = jnp.swapaxes(xf, 1, 2)
    xt = jnp.pad(xt, ((0, 0), (0, 0), (0, cpad - c)))
    out = pl.pallas_call(
        functools.partial(_knn_kernel, n=n, k=_K),
        grid=(b,),
        in_specs=[pl.BlockSpec((None, n, cpad), lambda i: (i, 0, 0))],
        out_specs=pl.BlockSpec((None, n, _KPAD), lambda i: (i, 0, 0)),
        out_shape=jax.ShapeDtypeStruct((b, n, _KPAD), jnp.int32),
    )(xt)
    return out[:, :, :_K]


def _vn_linear(w, x):
    return jnp.moveaxis(jnp.tensordot(w, x, axes=([1], [1])), 0, 1)


def _vn_batchnorm(x, dim):
    norm = jnp.linalg.norm(x, axis=2) + _EPS
    if dim == 5:
        axes = (0, 2, 3)
    elif dim == 4:
        axes = (0, 2)
    else:
        axes = (0,)
    mean = jnp.mean(norm, axis=axes, keepdims=True)
    var = jnp.mean((norm - mean) ** 2, axis=axes, keepdims=True)
    norm_bn = (norm - mean) / jnp.sqrt(var + _BN_EPS)
    return x / jnp.expand_dims(norm, 2) * jnp.expand_dims(norm_bn, 2)


def _vn_lrelu(w, wd, x, dim, slope=0.2):
    p = _vn_batchnorm(_vn_linear(w, x), dim)
    d = _vn_linear(wd, x)
    dot = jnp.sum(p * d, axis=2, keepdims=True)
    mask = (dot >= 0).astype(x.dtype)
    dns = jnp.sum(d * d, axis=2, keepdims=True)
    return slope * p + (1.0 - slope) * (
        mask * p + (1.0 - mask) * (p - (dot / (dns + _EPS)) * d)
    )


def _graph_feature(x, k):
    b, n = x.shape[0], x.shape[3]
    xf = x.reshape(b, -1, n)
    idx = _knn_idx(xf)
    c = xf.shape[1] // 3
    xt = jnp.swapaxes(xf, 2, 1)
    feat = xt[jnp.arange(b)[:, None, None], idx].reshape(b, n, k, c, 3)
    xr = jnp.broadcast_to(xt.reshape(b, n, 1, c, 3), (b, n, k, c, 3))
    feat = jnp.concatenate([feat - xr, xr], axis=3)
    return jnp.transpose(feat, (0, 3, 4, 1, 2))


def kernel(x, c1W, c1D, c2W, c2D, c3W, c3D, c4W, c4D, c5W, c5D, l1W, l1D, l2W, l2D, l3W):
    h = x[:, None, :, :]
    h = _vn_lrelu(c1W, c1D, _graph_feature(h, _K), 5)
    x1 = jnp.mean(h, axis=-1)
    h = _vn_lrelu(c2W, c2D, _graph_feature(x1, _K), 5)
    x2 = jnp.mean(h, axis=-1)
    h = _vn_lrelu(c3W, c3D, _graph_feature(x2, _K), 5)
    x3 = jnp.mean(h, axis=-1)
    h = _vn_lrelu(c4W, c4D, _graph_feature(x3, _K), 5)
    x4 = jnp.mean(h, axis=-1)
    h = jnp.concatenate([x1, x2, x3, x4], axis=1)
    h = _vn_lrelu(c5W, c5D, h, 4)
    hm = jnp.broadcast_to(jnp.mean(h, axis=-1, keepdims=True), h.shape)
    h = jnp.concatenate([h, hm], axis=1)
    h = jnp.mean(h, axis=-1)
    h = _vn_lrelu(l1W, l1D, h, 3)
    h = _vn_lrelu(l2W, l2D, h, 3)
    h = _vn_linear(l3W, h)
    return jnp.swapaxes(h, -1, -2)
